# Initial kernel scaffold; baseline (speedup 1.0000x reference)
import jax
import jax.numpy as jnp
from jax.experimental import pallas as pl


def kernel(x, edge_index, W1, b1, W2, b2):
    raise NotImplementedError("write your pallas kernel here")


# speedup vs baseline: 8.9396x; 8.9396x over previous
import functools

import jax
import jax.numpy as jnp
from jax import lax
from jax.experimental import pallas as pl
from jax.experimental.pallas import tpu as pltpu
from jax.experimental.pallas import tpu_sc as plsc

NC = 2
NS = 16
NW = NC * NS
CHUNK = 128
DW = 128

_mesh = plsc.VectorSubcoreMesh(core_axis_name="c", subcore_axis_name="s")


def _make_deg_kernel(np_rows, cpt):
    rpt = np_rows // NS

    @functools.partial(
        pl.kernel,
        out_type=jax.ShapeDtypeStruct((NC, np_rows, DW), jnp.float32),
        mesh=_mesh,
        scratch_types=[
            pltpu.VMEM((CHUNK,), jnp.int32),
            pltpu.VMEM((CHUNK, DW), jnp.float32),
            pltpu.VMEM_SHARED((np_rows, DW), jnp.float32),
            pltpu.SemaphoreType.DMA,
        ],
    )
    def deg_kernel(dstp_hbm, ones_hbm, zeros_hbm, out_hbm, idx_v, ones_v, acc_sh, sem):
        c = lax.axis_index("c")
        s = lax.axis_index("s")
        wid = s * NC + c
        base = wid * cpt
        pltpu.sync_copy(zeros_hbm.at[pl.ds(s * rpt, rpt)], acc_sh.at[pl.ds(s * rpt, rpt)])
        pltpu.sync_copy(ones_hbm, ones_v)
        plsc.subcore_barrier()

        def body(j, carry):
            pltpu.sync_copy(dstp_hbm.at[pl.ds((base + j) * CHUNK, CHUNK)], idx_v)
            pltpu.sync_copy(ones_v, acc_sh.at[idx_v], add=True)
            return carry

        lax.fori_loop(0, cpt, body, 0)
        plsc.subcore_barrier()
        pltpu.sync_copy(acc_sh.at[pl.ds(s * rpt, rpt)], out_hbm.at[c, pl.ds(s * rpt, rpt)])

    return deg_kernel


def _make_agg_kernel(np_rows, cpt, d):
    rpt = np_rows // NS

    @functools.partial(
        pl.kernel,
        out_type=jax.ShapeDtypeStruct((NC, np_rows, d), jnp.float32),
        mesh=_mesh,
        scratch_types=[
            pltpu.VMEM((CHUNK,), jnp.int32),
            pltpu.VMEM((CHUNK,), jnp.int32),
            pltpu.VMEM((CHUNK,), jnp.int32),
            pltpu.VMEM((CHUNK, d), jnp.float32),
            pltpu.VMEM((CHUNK, d), jnp.float32),
            pltpu.VMEM_SHARED((np_rows, d), jnp.float32),
            pltpu.SemaphoreType.DMA,
            pltpu.SemaphoreType.DMA,
        ],
    )
    def agg_kernel(h_hbm, srcp_hbm, dstp_hbm, zeros_hbm, out_hbm,
                   src0, src1, didx, buf0, buf1, acc_sh, sem0, sem1):
        c = lax.axis_index("c")
        s = lax.axis_index("s")
        wid = s * NC + c
        base = wid * cpt
        pltpu.sync_copy(zeros_hbm.at[pl.ds(s * rpt, rpt)], acc_sh.at[pl.ds(s * rpt, rpt)])
        plsc.subcore_barrier()

        pltpu.sync_copy(srcp_hbm.at[pl.ds(base * CHUNK, CHUNK)], src0)
        pltpu.async_copy(h_hbm.at[src0], buf0, sem0)
        pltpu.sync_copy(srcp_hbm.at[pl.ds((base + 1) * CHUNK, CHUNK)], src1)
        pltpu.async_copy(h_hbm.at[src1], buf1, sem1)

        def body(t, carry):
            j = 2 * t
            pltpu.make_async_copy(h_hbm.at[src0], buf0, sem0).wait()
            pltpu.sync_copy(dstp_hbm.at[pl.ds((base + j) * CHUNK, CHUNK)], didx)
            pltpu.sync_copy(buf0, acc_sh.at[didx], add=True)

            @pl.when(j + 2 < cpt)
            def _():
                pltpu.sync_copy(srcp_hbm.at[pl.ds((base + j + 2) * CHUNK, CHUNK)], src0)
                pltpu.async_copy(h_hbm.at[src0], buf0, sem0)

            pltpu.make_async_copy(h_hbm.at[src1], buf1, sem1).wait()
            pltpu.sync_copy(dstp_hbm.at[pl.ds((base + j + 1) * CHUNK, CHUNK)], didx)
            pltpu.sync_copy(buf1, acc_sh.at[didx], add=True)

            @pl.when(j + 3 < cpt)
            def _():
                pltpu.sync_copy(srcp_hbm.at[pl.ds((base + j + 3) * CHUNK, CHUNK)], src1)
                pltpu.async_copy(h_hbm.at[src1], buf1, sem1)

            return carry

        lax.fori_loop(0, cpt // 2, body, 0)
        plsc.subcore_barrier()
        pltpu.sync_copy(acc_sh.at[pl.ds(s * rpt, rpt)], out_hbm.at[c, pl.ds(s * rpt, rpt)])

    return agg_kernel


def _tc_first(deg2, x, w1, br):
    n, d = x.shape

    def body(deg_ref, x_ref, w_ref, h_ref, d_ref):
        deg = deg_ref[0, :, :1] + deg_ref[1, :, :1] + 1.0
        dis = lax.rsqrt(deg)
        h_ref[...] = jnp.dot(x_ref[...] * dis, w_ref[...],
                             preferred_element_type=jnp.float32)
        d_ref[...] = jnp.broadcast_to(dis, (br, 8))

    return pl.pallas_call(
        body,
        grid=(n // br,),
        in_specs=[
            pl.BlockSpec((NC, br, DW), lambda i: (0, i, 0)),
            pl.BlockSpec((br, d), lambda i: (i, 0)),
            pl.BlockSpec((d, d), lambda i: (0, 0)),
        ],
        out_specs=[
            pl.BlockSpec((br, d), lambda i: (i, 0)),
            pl.BlockSpec((br, 8), lambda i: (i, 0)),
        ],
        out_shape=[
            jax.ShapeDtypeStruct((n, d), jnp.float32),
            jax.ShapeDtypeStruct((n, 8), jnp.float32),
        ],
    )(deg2, x, w1)


def _tc_mid(acc, h1s, d8, b1, w2, br):
    n, d = h1s.shape

    def body2(acc_ref, h_ref, d_ref, b_ref, w_ref, out_ref):
        agg = acc_ref[0] + acc_ref[1] + h_ref[...]
        dis = d_ref[:, :1]
        t = jnp.maximum(agg * dis + b_ref[...], 0.0) * dis
        out_ref[...] = jnp.dot(t, w_ref[...], preferred_element_type=jnp.float32)

    return pl.pallas_call(
        body2,
        grid=(n // br,),
        in_specs=[
            pl.BlockSpec((NC, br, d), lambda i: (0, i, 0)),
            pl.BlockSpec((br, d), lambda i: (i, 0)),
            pl.BlockSpec((br, 8), lambda i: (i, 0)),
            pl.BlockSpec((1, d), lambda i: (0, 0)),
            pl.BlockSpec((d, d), lambda i: (0, 0)),
        ],
        out_specs=pl.BlockSpec((br, d), lambda i: (i, 0)),
        out_shape=jax.ShapeDtypeStruct((n, d), jnp.float32),
    )(acc, h1s, d8, b1, w2)


def _tc_last(acc, h2s, d8, b2, br):
    n, d = h2s.shape

    def body(acc_ref, h_ref, d_ref, b_ref, out_ref):
        agg = acc_ref[0] + acc_ref[1] + h_ref[...]
        out_ref[...] = agg * d_ref[:, :1] + b_ref[...]

    return pl.pallas_call(
        body,
        grid=(n // br,),
        in_specs=[
            pl.BlockSpec((NC, br, d), lambda i: (0, i, 0)),
            pl.BlockSpec((br, d), lambda i: (i, 0)),
            pl.BlockSpec((br, 8), lambda i: (i, 0)),
            pl.BlockSpec((1, d), lambda i: (0, 0)),
        ],
        out_specs=pl.BlockSpec((br, d), lambda i: (i, 0)),
        out_shape=jax.ShapeDtypeStruct((n, d), jnp.float32),
    )(acc, h2s, d8, b2)


def kernel(x, edge_index, W1, b1, W2, b2):
    n, d = x.shape
    e = edge_index.shape[1]
    cpt = -(-(-(-e // (NW * CHUNK))) // 2) * 2
    ep = NW * CHUNK * cpt
    np_rows = -(-(n + 1) // (NS * 8)) * (NS * 8)
    scratch_row = n
    br = 400

    ei = edge_index.astype(jnp.int32)
    pad = ep - e
    srcp = jnp.concatenate([ei[0], jnp.zeros((pad,), jnp.int32)])
    dstp = jnp.concatenate([ei[1], jnp.full((pad,), scratch_row, jnp.int32)])

    zeros = jnp.zeros((np_rows, d), jnp.float32)
    zeros8 = jnp.zeros((np_rows, DW), jnp.float32)
    ones8 = jnp.ones((CHUNK, DW), jnp.float32)

    deg_k = _make_deg_kernel(np_rows, cpt)
    agg_k = _make_agg_kernel(np_rows, cpt, d)

    deg2 = deg_k(dstp, ones8, zeros8)
    h1s, d8 = _tc_first(deg2, x, W1, br)
    acc1 = agg_k(h1s, srcp, dstp, zeros)
    h2s = _tc_mid(acc1, h1s, d8, b1.reshape(1, d), W2, br)
    acc2 = agg_k(h2s, srcp, dstp, zeros)
    out = _tc_last(acc2, h2s, d8, b2.reshape(1, d), br)
    return out
